# baseline (speedup 1.0000x reference)
import functools

import jax
import jax.numpy as jnp
from jax import lax
from jax.experimental import pallas as pl
from jax.experimental.pallas import tpu as pltpu
from jax.experimental.pallas import tpu_sc as plsc

_N_LOC = 49
_N_SA = 196
_B = 16384
_DP = 208
_NC = 2
_NS = 16
_NW = _NC * _NS
_BPW = _B // _NW
_CHUNK = 128
_NCHUNK = _BPW // _CHUNK
_L = 16
_NG = _BPW // _L
_GSZ = _DP * _L


def _make_noise():
    g = jnp.pad(
        jax.random.gumbel(jax.random.key(42), (_B, _N_SA), jnp.float32),
        ((0, 0), (0, _DP - _N_SA)))
    g = g.reshape(_NW * _NG, _L, _DP).transpose(0, 2, 1)
    return g.reshape(-1)


_G_FLAT = _make_noise()


def _tc_log_body(t_ref, o_ref):
    x = t_ref[0]
    y = jnp.log(x + 1e-12)
    o_ref[0] = jnp.concatenate(
        [y, jnp.zeros((_N_SA, _DP - _N_SA), jnp.float32)], axis=1)


def _tc_log_table(T):
    return pl.pallas_call(
        _tc_log_body,
        grid=(_N_LOC,),
        in_specs=[pl.BlockSpec((1, _N_SA, _N_SA), lambda l: (l, 0, 0))],
        out_specs=pl.BlockSpec((1, _N_SA, _DP), lambda l: (l, 0, 0)),
        out_shape=jax.ShapeDtypeStruct((_N_LOC, _N_SA, _DP), jnp.float32),
    )(T).reshape(_N_LOC * _N_SA, _DP)


def _sc_sample(logt_pad, state, reward, g_flat):
    mesh = plsc.VectorSubcoreMesh(core_axis_name="c", subcore_axis_name="s")

    @functools.partial(
        pl.kernel,
        mesh=mesh,
        compiler_params=pltpu.CompilerParams(use_tc_tiling_on_sc=False,
                                             needs_layout_passes=False),
        out_type=(
            jax.ShapeDtypeStruct((_B,), jnp.int32),
            jax.ShapeDtypeStruct((_B,), jnp.int32),
        ),
        scratch_types=[
            pltpu.VMEM((_BPW,), jnp.int32),
            pltpu.VMEM((_BPW,), jnp.int32),
            pltpu.VMEM((_BPW,), jnp.int32),
            pltpu.VMEM((2, _CHUNK, _DP), jnp.float32),
            pltpu.VMEM((2, _CHUNK // _L * _GSZ), jnp.float32),
            pltpu.VMEM((_BPW,), jnp.int32),
            pltpu.VMEM((_BPW,), jnp.int32),
            pltpu.SemaphoreType.DMA,
            pltpu.SemaphoreType.DMA,
        ],
    )
    def k(logt_hbm, state_hbm, reward_hbm, g_hbm, sa_hbm, term_hbm,
          st_v, rw_v, idx_v, rows_v, g_v, sa_v, term_v, sem_r, sem_g):
        wid = lax.axis_index("s") * _NC + lax.axis_index("c")
        base = wid * _BPW
        pltpu.sync_copy(state_hbm.at[pl.ds(base, _BPW)], st_v)
        pltpu.sync_copy(reward_hbm.at[pl.ds(base, _BPW)], rw_v)
        for i in range(_BPW // _L):
            s = pl.ds(i * _L, _L)
            idx_v[s] = rw_v[s] * _N_SA + st_v[s]

        gpc = _CHUNK // _L

        def fire(c, buf):
            cp_r = pltpu.async_copy(
                logt_hbm.at[idx_v.at[pl.ds(c * _CHUNK, _CHUNK)]],
                rows_v.at[buf], sem_r)
            goff = (wid * _NG + c * gpc) * _GSZ
            cp_g = pltpu.async_copy(
                g_hbm.at[pl.ds(goff, gpc * _GSZ)], g_v.at[buf], sem_g)
            return cp_r, cp_g

        iota = lax.iota(jnp.int32, _L)
        ninf = jnp.full((_L,), -jnp.inf, jnp.float32)
        zero = jnp.zeros((_L,), jnp.int32)
        pend = fire(0, 0)
        for c in range(_NCHUNK):
            buf = c % 2
            if c + 1 < _NCHUNK:
                nxt = fire(c + 1, (c + 1) % 2)
            pend[0].wait()
            pend[1].wait()
            if c + 1 < _NCHUNK:
                pend = nxt
            rows_ref = rows_v.at[buf]
            g_ref = g_v.at[buf]
            for j in range(gpc):
                row_ids = iota + (j * _L)
                gbase = j * _GSZ

                def kbody(i, carry, _gbase=gbase, _rows=rows_ref, _g=g_ref,
                          _rows_ids=row_ids):
                    vmax0, varg0, vmax1, varg1 = carry
                    k0 = i * 4
                    for u in (0, 1):
                        col = lax.broadcast(k0 + u, (_L,))
                        v = (plsc.load_gather(_rows, [_rows_ids, col])
                             + _g[pl.ds(_gbase + (k0 + u) * _L, _L)])
                        m = v > vmax0
                        vmax0 = jnp.where(m, v, vmax0)
                        varg0 = jnp.where(m, col, varg0)
                    for u in (2, 3):
                        col = lax.broadcast(k0 + u, (_L,))
                        v = (plsc.load_gather(_rows, [_rows_ids, col])
                             + _g[pl.ds(_gbase + (k0 + u) * _L, _L)])
                        m = v > vmax1
                        vmax1 = jnp.where(m, v, vmax1)
                        varg1 = jnp.where(m, col, varg1)
                    return (vmax0, varg0, vmax1, varg1)

                vmax0, varg0, vmax1, varg1 = lax.fori_loop(
                    0, _N_SA // 4, kbody, (ninf, zero, ninf, zero))
                take1 = (vmax1 > vmax0) | ((vmax1 == vmax0) & (varg1 < varg0))
                sa16 = jnp.where(take1, varg1, varg0)
                o = pl.ds(c * _CHUNK + j * _L, _L)
                rw16 = rw_v[o]
                sa_v[o] = sa16
                term_v[o] = jnp.where(lax.rem(sa16, _N_LOC) == rw16, 1, 0)
        pltpu.sync_copy(sa_v, sa_hbm.at[pl.ds(base, _BPW)])
        pltpu.sync_copy(term_v, term_hbm.at[pl.ds(base, _BPW)])

    return k(logt_pad, state, reward, g_flat)


def kernel(state, reward, T):
    state = state.astype(jnp.int32)
    reward = reward.astype(jnp.int32)
    sa, term = _sc_sample(_tc_log_table(T), state, reward, _G_FLAT)
    return sa, term.astype(jnp.bool_)


# speedup vs baseline: 1.0095x; 1.0095x over previous
import functools

import jax
import jax.numpy as jnp
from jax import lax
from jax.experimental import pallas as pl
from jax.experimental.pallas import tpu as pltpu
from jax.experimental.pallas import tpu_sc as plsc

_N_LOC = 49
_N_SA = 196
_B = 16384
_DP = 208
_NC = 2
_NS = 16
_NW = _NC * _NS
_BPW = _B // _NW
_CHUNK = 128
_NCHUNK = _BPW // _CHUNK
_L = 16
_NG = _BPW // _L
_GSZ = _DP * _L


def _make_noise():
    g = jnp.pad(
        jax.random.gumbel(jax.random.key(42), (_B, _N_SA), jnp.float32),
        ((0, 0), (0, _DP - _N_SA)))
    g = g.reshape(_NW * _NG, _L, _DP).transpose(0, 2, 1)
    return g.reshape(-1)


_G_FLAT = _make_noise()


def _log_table(T):
    return jnp.pad(
        jnp.log(T.reshape(_N_LOC * _N_SA, _N_SA) + 1e-12),
        ((0, 0), (0, _DP - _N_SA)))


def _sc_sample(logt_pad, state, reward, g_flat):
    mesh = plsc.VectorSubcoreMesh(core_axis_name="c", subcore_axis_name="s")

    @functools.partial(
        pl.kernel,
        mesh=mesh,
        compiler_params=pltpu.CompilerParams(use_tc_tiling_on_sc=False,
                                             needs_layout_passes=False),
        out_type=(
            jax.ShapeDtypeStruct((_B,), jnp.int32),
            jax.ShapeDtypeStruct((_B,), jnp.int32),
        ),
        scratch_types=[
            pltpu.VMEM((_BPW,), jnp.int32),
            pltpu.VMEM((_BPW,), jnp.int32),
            pltpu.VMEM((_BPW,), jnp.int32),
            pltpu.VMEM((2, _CHUNK, _DP), jnp.float32),
            pltpu.VMEM((2, _CHUNK // _L * _GSZ), jnp.float32),
            pltpu.VMEM((_BPW,), jnp.int32),
            pltpu.VMEM((_BPW,), jnp.int32),
            pltpu.SemaphoreType.DMA,
            pltpu.SemaphoreType.DMA,
        ],
    )
    def k(logt_hbm, state_hbm, reward_hbm, g_hbm, sa_hbm, term_hbm,
          st_v, rw_v, idx_v, rows_v, g_v, sa_v, term_v, sem_r, sem_g):
        wid = lax.axis_index("s") * _NC + lax.axis_index("c")
        base = wid * _BPW
        pltpu.sync_copy(state_hbm.at[pl.ds(base, _BPW)], st_v)
        pltpu.sync_copy(reward_hbm.at[pl.ds(base, _BPW)], rw_v)
        for i in range(_BPW // _L):
            s = pl.ds(i * _L, _L)
            idx_v[s] = rw_v[s] * _N_SA + st_v[s]

        gpc = _CHUNK // _L

        def fire(c, buf):
            cp_r = pltpu.async_copy(
                logt_hbm.at[idx_v.at[pl.ds(c * _CHUNK, _CHUNK)]],
                rows_v.at[buf], sem_r)
            goff = (wid * _NG + c * gpc) * _GSZ
            cp_g = pltpu.async_copy(
                g_hbm.at[pl.ds(goff, gpc * _GSZ)], g_v.at[buf], sem_g)
            return cp_r, cp_g

        iota = lax.iota(jnp.int32, _L)
        ninf = jnp.full((_L,), -jnp.inf, jnp.float32)
        zero = jnp.zeros((_L,), jnp.int32)
        pend = fire(0, 0)
        for c in range(_NCHUNK):
            buf = c % 2
            if c + 1 < _NCHUNK:
                nxt = fire(c + 1, (c + 1) % 2)
            pend[0].wait()
            pend[1].wait()
            if c + 1 < _NCHUNK:
                pend = nxt
            rows_ref = rows_v.at[buf]
            g_ref = g_v.at[buf]
            for j in range(gpc):
                row_ids = iota + (j * _L)
                gbase = j * _GSZ

                def step(u, k, acc, _gbase=gbase, _rows=rows_ref, _g=g_ref,
                         _row_ids=row_ids):
                    col = lax.broadcast(k, (_L,))
                    v = (plsc.load_gather(_rows, [_row_ids, col])
                         + _g[pl.ds(_gbase + k * _L, _L)])
                    m = v > acc[u]
                    acc[u] = jnp.where(m, v, acc[u])
                    acc[8 + u] = jnp.where(m, col, acc[8 + u])

                def kbody(i, carry):
                    acc = list(carry)
                    for u in range(8):
                        step(u, i * 8 + u, acc)
                    return tuple(acc)

                res = list(lax.fori_loop(0, _N_SA // 8, kbody,
                                         (ninf,) * 8 + (zero,) * 8))
                for u in range(_N_SA % 8):
                    step(u, (_N_SA // 8) * 8 + u, res)
                vmax0, varg0 = res[0], res[8]
                for q in range(1, 8):
                    vm, va = res[q], res[8 + q]
                    takeq = (vm > vmax0) | ((vm == vmax0) & (va < varg0))
                    vmax0 = jnp.where(takeq, vm, vmax0)
                    varg0 = jnp.where(takeq, va, varg0)
                sa16 = varg0
                o = pl.ds(c * _CHUNK + j * _L, _L)
                rw16 = rw_v[o]
                sa_v[o] = sa16
                term_v[o] = jnp.where(lax.rem(sa16, _N_LOC) == rw16, 1, 0)
        pltpu.sync_copy(sa_v, sa_hbm.at[pl.ds(base, _BPW)])
        pltpu.sync_copy(term_v, term_hbm.at[pl.ds(base, _BPW)])

    return k(logt_pad, state, reward, g_flat)


def kernel(state, reward, T):
    state = state.astype(jnp.int32)
    reward = reward.astype(jnp.int32)
    sa, term = _sc_sample(_log_table(T), state, reward, _G_FLAT)
    return sa, term.astype(jnp.bool_)


# speedup vs baseline: 1.4342x; 1.4207x over previous
import functools

import jax
import jax.numpy as jnp
from jax import lax
from jax.experimental import pallas as pl
from jax.experimental.pallas import tpu as pltpu
from jax.experimental.pallas import tpu_sc as plsc

_N_LOC = 49
_N_SA = 196
_B = 16384
_DP = 208
_NC = 2
_NS = 16
_NW = _NC * _NS
_BPW = _B // _NW
_CHUNK = 128
_NCHUNK = _BPW // _CHUNK
_L = 16
_NG = _BPW // _L
_GSZ = _DP * _L


def _make_noise():
    g = jnp.pad(
        jax.random.gumbel(jax.random.key(42), (_B, _N_SA), jnp.float32),
        ((0, 0), (0, _DP - _N_SA)))
    g = g.reshape(_NW * _NG, _L, _DP).transpose(0, 2, 1)
    return g.reshape(-1)


_G_FLAT = _make_noise()


def _log_table(T):
    return jnp.pad(jnp.log(T + 1e-12),
                   ((0, 0), (0, 0), (0, _DP - _N_SA))
                   ).reshape(_N_LOC * _N_SA, _DP)


def _sc_sample(logt_pad, state, reward, g_flat):
    mesh = plsc.VectorSubcoreMesh(core_axis_name="c", subcore_axis_name="s")

    @functools.partial(
        pl.kernel,
        mesh=mesh,
        compiler_params=pltpu.CompilerParams(use_tc_tiling_on_sc=False,
                                             needs_layout_passes=False),
        out_type=(
            jax.ShapeDtypeStruct((_B,), jnp.int32),
            jax.ShapeDtypeStruct((_B,), jnp.int32),
        ),
        scratch_types=[
            pltpu.VMEM((_BPW,), jnp.int32),
            pltpu.VMEM((_BPW,), jnp.int32),
            pltpu.VMEM((_BPW,), jnp.int32),
            pltpu.VMEM((2, _CHUNK, _DP), jnp.float32),
            pltpu.VMEM((2, _CHUNK // _L, _DP, _L), jnp.float32),
            pltpu.VMEM((_BPW,), jnp.int32),
            pltpu.VMEM((_BPW,), jnp.int32),
            pltpu.SemaphoreType.DMA,
            pltpu.SemaphoreType.DMA,
        ],
    )
    def k(logt_hbm, state_hbm, reward_hbm, g_hbm, sa_hbm, term_hbm,
          st_v, rw_v, idx_v, rows_v, g_v, sa_v, term_v, sem_r, sem_g):
        wid = lax.axis_index("s") * _NC + lax.axis_index("c")
        base = wid * _BPW
        pltpu.sync_copy(state_hbm.at[pl.ds(base, _BPW)], st_v)
        pltpu.sync_copy(reward_hbm.at[pl.ds(base, _BPW)], rw_v)
        for i in range(_BPW // _L):
            s = pl.ds(i * _L, _L)
            idx_v[s] = rw_v[s] * _N_SA + st_v[s]

        gpc = _CHUNK // _L

        def fire(c, buf):
            cp_r = pltpu.async_copy(
                logt_hbm.at[idx_v.at[pl.ds(c * _CHUNK, _CHUNK)]],
                rows_v.at[buf], sem_r)
            cp_g = pltpu.async_copy(
                g_hbm.at[pl.ds(wid * _NG + c * gpc, gpc)], g_v.at[buf], sem_g)
            return cp_r, cp_g

        iota = lax.iota(jnp.int32, _L)
        ninf = jnp.full((_L,), -jnp.inf, jnp.float32)
        zero = jnp.zeros((_L,), jnp.int32)
        pend = fire(0, 0)
        for c in range(_NCHUNK):
            buf = c % 2
            if c + 1 < _NCHUNK:
                nxt = fire(c + 1, (c + 1) % 2)
            pend[0].wait()
            pend[1].wait()
            if c + 1 < _NCHUNK:
                pend = nxt
            rows_ref = rows_v.at[buf]
            for j in range(gpc):
                row_ids = iota + (j * _L)
                g_ref = g_v.at[buf, j]

                def step(u, k, acc, _rows=rows_ref, _g=g_ref,
                         _row_ids=row_ids):
                    col = lax.broadcast(k, (_L,))
                    v = plsc.load_gather(_rows, [_row_ids, col]) + _g[k]
                    m = v > acc[u]
                    acc[u] = jnp.where(m, v, acc[u])
                    acc[8 + u] = jnp.where(m, col, acc[8 + u])

                def kbody(i, carry):
                    acc = list(carry)
                    for u in range(8):
                        step(u, i * 8 + u, acc)
                    return tuple(acc)

                res = list(lax.fori_loop(0, _N_SA // 8, kbody,
                                         (ninf,) * 8 + (zero,) * 8))
                for u in range(_N_SA % 8):
                    step(u, (_N_SA // 8) * 8 + u, res)
                vmax0, varg0 = res[0], res[8]
                for q in range(1, 8):
                    vm, va = res[q], res[8 + q]
                    takeq = (vm > vmax0) | ((vm == vmax0) & (va < varg0))
                    vmax0 = jnp.where(takeq, vm, vmax0)
                    varg0 = jnp.where(takeq, va, varg0)
                sa16 = varg0
                o = pl.ds(c * _CHUNK + j * _L, _L)
                rw16 = rw_v[o]
                sa_v[o] = sa16
                term_v[o] = jnp.where(lax.rem(sa16, _N_LOC) == rw16, 1, 0)
        pltpu.sync_copy(sa_v, sa_hbm.at[pl.ds(base, _BPW)])
        pltpu.sync_copy(term_v, term_hbm.at[pl.ds(base, _BPW)])

    return k(logt_pad, state, reward, g_flat)


def kernel(state, reward, T):
    state = state.astype(jnp.int32)
    reward = reward.astype(jnp.int32)
    g3 = _G_FLAT.reshape(_NW * _NG, _DP, _L)
    sa, term = _sc_sample(_log_table(T), state, reward, g3)
    return sa, term.astype(jnp.bool_)


# speedup vs baseline: 2.1444x; 1.4952x over previous
import functools

import jax
import jax.numpy as jnp
from jax import lax
from jax.experimental import pallas as pl
from jax.experimental.pallas import tpu as pltpu
from jax.experimental.pallas import tpu_sc as plsc

_N_LOC = 49
_N_SA = 196
_B = 16384
_DP = 208
_NC = 2
_NS = 16
_NW = _NC * _NS
_BPW = _B // _NW
_CHUNK = 128
_NCHUNK = _BPW // _CHUNK
_L = 16
_NG = _BPW // _L
_GSZ = _DP * _L


def _make_noise():
    g = jnp.pad(
        jax.random.gumbel(jax.random.key(42), (_B, _N_SA), jnp.float32),
        ((0, 0), (0, _DP - _N_SA)))
    g = g.reshape(_NW, _NG, _L, _DP // _L, _L).transpose(0, 1, 4, 3, 2)
    return g.reshape(-1)


_G_FLAT = _make_noise()


def _log_table(T):
    return jnp.pad(jnp.log(T + 1e-12),
                   ((0, 0), (0, 0), (0, _DP - _N_SA))
                   ).reshape(_N_LOC * _N_SA, _DP)


def _sc_sample(logt_pad, state, reward, g_flat):
    mesh = plsc.VectorSubcoreMesh(core_axis_name="c", subcore_axis_name="s")

    @functools.partial(
        pl.kernel,
        mesh=mesh,
        compiler_params=pltpu.CompilerParams(use_tc_tiling_on_sc=False,
                                             needs_layout_passes=False),
        out_type=(
            jax.ShapeDtypeStruct((_B,), jnp.int32),
            jax.ShapeDtypeStruct((_B,), jnp.int32),
        ),
        scratch_types=[
            pltpu.VMEM((_BPW,), jnp.int32),
            pltpu.VMEM((_BPW,), jnp.int32),
            pltpu.VMEM((_BPW,), jnp.int32),
            pltpu.VMEM((2, _CHUNK, _DP), jnp.float32),
            pltpu.VMEM((2, _CHUNK, _DP), jnp.float32),
            pltpu.VMEM((_BPW,), jnp.int32),
            pltpu.VMEM((_BPW,), jnp.int32),
            pltpu.SemaphoreType.DMA,
            pltpu.SemaphoreType.DMA,
        ],
    )
    def k(logt_hbm, state_hbm, reward_hbm, g_hbm, sa_hbm, term_hbm,
          st_v, rw_v, idx_v, rows_v, g_v, sa_v, term_v, sem_r, sem_g):
        wid = lax.axis_index("s") * _NC + lax.axis_index("c")
        base = wid * _BPW
        pltpu.sync_copy(state_hbm.at[pl.ds(base, _BPW)], st_v)
        pltpu.sync_copy(reward_hbm.at[pl.ds(base, _BPW)], rw_v)
        for i in range(_BPW // _L):
            s = pl.ds(i * _L, _L)
            idx_v[s] = rw_v[s] * _N_SA + st_v[s]

        gpc = _CHUNK // _L

        def fire(c, buf):
            cp_r = pltpu.async_copy(
                logt_hbm.at[idx_v.at[pl.ds(c * _CHUNK, _CHUNK)]],
                rows_v.at[buf], sem_r)
            cp_g = pltpu.async_copy(
                g_hbm.at[pl.ds((wid * _NG + c * gpc) * _L, _CHUNK)],
                g_v.at[buf], sem_g)
            return cp_r, cp_g

        iota = lax.iota(jnp.int32, _L)
        ninf = jnp.full((_L,), -jnp.inf, jnp.float32)
        zero = jnp.zeros((_L,), jnp.int32)
        pend = fire(0, 0)
        for c in range(_NCHUNK):
            buf = c % 2
            if c + 1 < _NCHUNK:
                nxt = fire(c + 1, (c + 1) % 2)
            pend[0].wait()
            pend[1].wait()
            if c + 1 < _NCHUNK:
                pend = nxt
            rows_ref = rows_v.at[buf]
            g_ref = g_v.at[buf]
            for j in range(gpc):
                row_ids = iota + (j * _L)

                def step(u, k, acc, _j=j, _rows=rows_ref, _g=g_ref,
                         _row_ids=row_ids):
                    col = lax.broadcast(k, (_L,))
                    gn = _g[_j * _L + jnp.bitwise_and(k, _L - 1),
                            pl.ds(jnp.bitwise_and(k, -_L), _L)]
                    v = plsc.load_gather(_rows, [_row_ids, col]) + gn
                    m = v > acc[u]
                    acc[u] = jnp.where(m, v, acc[u])
                    acc[8 + u] = jnp.where(m, col, acc[8 + u])

                def kbody(i, carry):
                    acc = list(carry)
                    for u in range(8):
                        step(u, i * 8 + u, acc)
                    return tuple(acc)

                res = list(lax.fori_loop(0, _N_SA // 8, kbody,
                                         (ninf,) * 8 + (zero,) * 8))
                for u in range(_N_SA % 8):
                    step(u, (_N_SA // 8) * 8 + u, res)
                vmax0, varg0 = res[0], res[8]
                for q in range(1, 8):
                    vm, va = res[q], res[8 + q]
                    takeq = (vm > vmax0) | ((vm == vmax0) & (va < varg0))
                    vmax0 = jnp.where(takeq, vm, vmax0)
                    varg0 = jnp.where(takeq, va, varg0)
                sa16 = varg0
                o = pl.ds(c * _CHUNK + j * _L, _L)
                rw16 = rw_v[o]
                sa_v[o] = sa16
                term_v[o] = jnp.where(lax.rem(sa16, _N_LOC) == rw16, 1, 0)
        pltpu.sync_copy(sa_v, sa_hbm.at[pl.ds(base, _BPW)])
        pltpu.sync_copy(term_v, term_hbm.at[pl.ds(base, _BPW)])

    return k(logt_pad, state, reward, g_flat)


def kernel(state, reward, T):
    state = state.astype(jnp.int32)
    reward = reward.astype(jnp.int32)
    g2 = _G_FLAT.reshape(_B, _DP)
    sa, term = _sc_sample(_log_table(T), state, reward, g2)
    return sa, term.astype(jnp.bool_)
